# Initial kernel scaffold; baseline (speedup 1.0000x reference)
import jax
import jax.numpy as jnp
from jax.experimental import pallas as pl


def kernel(x, adj, W_heads, a_heads, W_out, a_out):
    raise NotImplementedError("write your pallas kernel here")


# speedup vs baseline: 3493.0593x; 3493.0593x over previous
import jax
import jax.numpy as jnp
from jax.experimental import pallas as pl

N = 1024
NFEAT = 128
NHID = 16
NOUT = 128
NHEADS = 8
ALPHA = 0.2


def _edge_weights(p_col, q_row, adj):
    s = p_col + q_row
    return adj * jnp.exp(-jnp.maximum(s, ALPHA * s))


def _gat_kernel(x_ref, adj_ref, wall_ref, a1_ref, a2_ref, wout_ref, ao_ref,
                out_ref):
    x = x_ref[...]
    adj = adj_ref[...]

    h_all = jnp.dot(x, wall_ref[...], preferred_element_type=jnp.float32)
    head_outs = []
    for i in range(NHEADS):
        h_i = h_all[:, i * NHID:(i + 1) * NHID]
        a1 = a1_ref[i:i + 1, :]
        a2 = a2_ref[i:i + 1, :]
        p = jax.lax.dot_general(h_i, a1, (((1,), (1,)), ((), ())),
                                preferred_element_type=jnp.float32)
        q = jax.lax.dot_general(a2, h_i, (((1,), (1,)), ((), ())),
                                preferred_element_type=jnp.float32)
        e = _edge_weights(p, q, adj)
        numer = jnp.dot(e, h_i, preferred_element_type=jnp.float32)
        denom = jnp.sum(e, axis=1, keepdims=True)
        hp = numer / denom
        head_outs.append(jnp.where(hp > 0, hp, jnp.exp(hp) - 1.0))
    x2 = jnp.concatenate(head_outs, axis=1)

    h2 = jnp.dot(x2, wout_ref[...], preferred_element_type=jnp.float32)
    a1o = ao_ref[:, :NOUT]
    a2o = ao_ref[:, NOUT:]
    p2 = jax.lax.dot_general(h2, a1o, (((1,), (1,)), ((), ())),
                             preferred_element_type=jnp.float32)
    q2 = jax.lax.dot_general(a2o, h2, (((1,), (1,)), ((), ())),
                             preferred_element_type=jnp.float32)
    e2 = _edge_weights(p2, q2, adj)
    numer2 = jnp.dot(e2, h2, preferred_element_type=jnp.float32)
    denom2 = jnp.sum(e2, axis=1, keepdims=True)
    h_out = numer2 / denom2

    deg = jnp.sum(adj, axis=1, keepdims=True)
    h_out = jnp.where(deg == 0.0, x, h_out)
    out_ref[...] = jnp.where(h_out > 0, h_out, jnp.exp(h_out) - 1.0)


def kernel(x, adj, W_heads, a_heads, W_out, a_out):
    w_all = jnp.transpose(W_heads, (1, 0, 2)).reshape(NFEAT, NHEADS * NHID)
    a1_all = a_heads[:, 0, :NHID]
    a2_all = a_heads[:, 0, NHID:]
    return pl.pallas_call(
        _gat_kernel,
        out_shape=jax.ShapeDtypeStruct((N, NOUT), jnp.float32),
    )(x, adj, w_all, a1_all, a2_all, W_out, a_out)


# speedup vs baseline: 3803.6294x; 1.0889x over previous
import jax
import jax.numpy as jnp
from jax.experimental import pallas as pl

N = 1024
NFEAT = 128
NHID = 16
NOUT = 128
NHEADS = 8
ALPHA = 0.2


def _edge_weights(p_col, q_row, adj):
    ab = jnp.exp(-p_col) * jnp.exp(-q_row)
    cd = jnp.exp(-ALPHA * p_col) * jnp.exp(-ALPHA * q_row)
    return adj * jnp.minimum(ab, cd)


def _agg(h, p, q, adj, ones_col):
    e = _edge_weights(p, q, adj)
    h_aug = jnp.concatenate([h, ones_col], axis=1)
    nd = jnp.dot(e, h_aug, preferred_element_type=jnp.float32)
    d = h.shape[1]
    return nd[:, :d] / nd[:, d:d + 1]


def _gat_kernel(x_ref, adj_ref, wall_ref, a1_ref, a2_ref, wout_ref, ao_ref,
                out_ref):
    x = x_ref[...]
    adj = adj_ref[...]
    ones_col = jnp.ones((N, 1), dtype=jnp.float32)

    h_all = jnp.dot(x, wall_ref[...], preferred_element_type=jnp.float32)
    head_outs = []
    for i in range(NHEADS):
        h_i = h_all[:, i * NHID:(i + 1) * NHID]
        a1 = a1_ref[i:i + 1, :]
        a2 = a2_ref[i:i + 1, :]
        p = jax.lax.dot_general(h_i, a1, (((1,), (1,)), ((), ())),
                                preferred_element_type=jnp.float32)
        q = jax.lax.dot_general(a2, h_i, (((1,), (1,)), ((), ())),
                                preferred_element_type=jnp.float32)
        hp = _agg(h_i, p, q, adj, ones_col)
        head_outs.append(jnp.where(hp > 0, hp, jnp.exp(hp) - 1.0))
    x2 = jnp.concatenate(head_outs, axis=1)

    h2 = jnp.dot(x2, wout_ref[...], preferred_element_type=jnp.float32)
    a1o = ao_ref[:, :NOUT]
    a2o = ao_ref[:, NOUT:]
    p2 = jax.lax.dot_general(h2, a1o, (((1,), (1,)), ((), ())),
                             preferred_element_type=jnp.float32)
    q2 = jax.lax.dot_general(a2o, h2, (((1,), (1,)), ((), ())),
                             preferred_element_type=jnp.float32)
    h_out = _agg(h2, p2, q2, adj, ones_col)

    deg = jnp.sum(adj, axis=1, keepdims=True)
    h_out = jnp.where(deg == 0.0, x, h_out)
    out_ref[...] = jnp.where(h_out > 0, h_out, jnp.exp(h_out) - 1.0)


def kernel(x, adj, W_heads, a_heads, W_out, a_out):
    w_all = jnp.transpose(W_heads, (1, 0, 2)).reshape(NFEAT, NHEADS * NHID)
    a1_all = a_heads[:, 0, :NHID]
    a2_all = a_heads[:, 0, NHID:]
    return pl.pallas_call(
        _gat_kernel,
        out_shape=jax.ShapeDtypeStruct((N, NOUT), jnp.float32),
    )(x, adj, w_all, a1_all, a2_all, W_out, a_out)


# speedup vs baseline: 3898.4508x; 1.0249x over previous
import jax
import jax.numpy as jnp
from jax.experimental import pallas as pl

N = 1024
NFEAT = 128
NHID = 16
NOUT = 128
NHEADS = 8
ALPHA = 0.2


def _agg(h, p, q, adj, ones_col):
    b = jnp.exp(-q)
    db = jnp.exp(-ALPHA * q)
    r = jnp.exp((1.0 - ALPHA) * p)
    e = adj * jnp.minimum(b, r * db)
    h_aug = jnp.concatenate([h, ones_col], axis=1)
    nd = jnp.dot(e, h_aug, preferred_element_type=jnp.float32)
    d = h.shape[1]
    return nd[:, :d] * (1.0 / nd[:, d:d + 1])


def _gat_kernel(x_ref, adj_ref, wall_ref, a1_ref, a2_ref, wout_ref, ao_ref,
                out_ref):
    x = x_ref[...]
    adj = adj_ref[...]
    ones_col = jnp.ones((N, 1), dtype=jnp.float32)

    h_all = jnp.dot(x, wall_ref[...], preferred_element_type=jnp.float32)
    head_outs = []
    for i in range(NHEADS):
        h_i = h_all[:, i * NHID:(i + 1) * NHID]
        a1 = a1_ref[i:i + 1, :]
        a2 = a2_ref[i:i + 1, :]
        p = jax.lax.dot_general(h_i, a1, (((1,), (1,)), ((), ())),
                                preferred_element_type=jnp.float32)
        q = jax.lax.dot_general(a2, h_i, (((1,), (1,)), ((), ())),
                                preferred_element_type=jnp.float32)
        hp = _agg(h_i, p, q, adj, ones_col)
        head_outs.append(jnp.where(hp > 0, hp, jnp.exp(hp) - 1.0))
    x2 = jnp.concatenate(head_outs, axis=1)

    h2 = jnp.dot(x2, wout_ref[...], preferred_element_type=jnp.float32)
    a1o = ao_ref[:, :NOUT]
    a2o = ao_ref[:, NOUT:]
    p2 = jax.lax.dot_general(h2, a1o, (((1,), (1,)), ((), ())),
                             preferred_element_type=jnp.float32)
    q2 = jax.lax.dot_general(a2o, h2, (((1,), (1,)), ((), ())),
                             preferred_element_type=jnp.float32)
    h_out = _agg(h2, p2, q2, adj, ones_col)

    deg = jnp.sum(adj, axis=1, keepdims=True)
    h_out = jnp.where(deg == 0.0, x, h_out)
    out_ref[...] = jnp.where(h_out > 0, h_out, jnp.exp(h_out) - 1.0)


def kernel(x, adj, W_heads, a_heads, W_out, a_out):
    w_all = jnp.transpose(W_heads, (1, 0, 2)).reshape(NFEAT, NHEADS * NHID)
    a1_all = a_heads[:, 0, :NHID]
    a2_all = a_heads[:, 0, NHID:]
    return pl.pallas_call(
        _gat_kernel,
        out_shape=jax.ShapeDtypeStruct((N, NOUT), jnp.float32),
    )(x, adj, w_all, a1_all, a2_all, W_out, a_out)
